# baseline (speedup 1.0000x reference)
import functools

import jax
import jax.numpy as jnp
from jax import lax
from jax.experimental import pallas as pl
from jax.experimental.pallas import tpu as pltpu
from jax.experimental.pallas import tpu_sc as plsc

SC_ROWS = 512
TC_BLK = 256


def _sc_body(p1_hbm, p2_hbm, n1_hbm, n2_hbm, d2_hbm, d1_hbm,
             p1_v, p2_v, n1_v, n2_v, d2_v, d1_v, *, rpw, n_neg, d_dim):
    nc = 2
    wid = lax.axis_index("s") * nc + lax.axis_index("c")
    base = wid * rpw

    pltpu.sync_copy(p1_hbm.at[pl.ds(base, rpw)], p1_v)
    pltpu.sync_copy(p2_hbm.at[pl.ds(base, rpw)], p2_v)

    nchunks = d_dim // 16

    def row_step(r, carry):
        row = base + r
        pltpu.sync_copy(n2_hbm.at[row], n2_v)
        pltpu.sync_copy(n1_hbm.at[row], n1_v)
        p1c = [p1_v.at[r][pl.ds(c * 16, 16)][...] for c in range(nchunks)]
        p2c = [p2_v.at[r][pl.ds(c * 16, 16)][...] for c in range(nchunks)]
        for n in range(n_neg):
            acc2 = p1c[0] * (n2_v.at[n][pl.ds(0, 16)][...] - p2c[0])
            acc1 = p2c[0] * (n1_v.at[n][pl.ds(0, 16)][...] - p1c[0])
            for c in range(1, nchunks):
                acc2 += p1c[c] * (n2_v.at[n][pl.ds(c * 16, 16)][...] - p2c[c])
                acc1 += p2c[c] * (n1_v.at[n][pl.ds(c * 16, 16)][...] - p1c[c])
            d2_v[n] = acc2
            d1_v[n] = acc1
        pltpu.sync_copy(d2_v, d2_hbm.at[row])
        pltpu.sync_copy(d1_v, d1_hbm.at[row])
        return carry

    lax.fori_loop(0, rpw, row_step, 0)


def _sc_scores(p1_feat, p2_feat, n1_feat, n2_feat):
    B, N, D = n1_feat.shape
    info = plsc.get_sparse_core_info()
    nw = info.num_cores * info.num_subcores
    rpw = SC_ROWS // nw
    mesh = plsc.VectorSubcoreMesh(core_axis_name="c", subcore_axis_name="s")
    kern = pl.kernel(
        functools.partial(_sc_body, rpw=rpw, n_neg=N, d_dim=D),
        out_type=(
            jax.ShapeDtypeStruct((SC_ROWS, N, 16), jnp.float32),
            jax.ShapeDtypeStruct((SC_ROWS, N, 16), jnp.float32),
        ),
        mesh=mesh,
        scratch_types=[
            pltpu.VMEM((rpw, D), jnp.float32),
            pltpu.VMEM((rpw, D), jnp.float32),
            pltpu.VMEM((N, D), jnp.float32),
            pltpu.VMEM((N, D), jnp.float32),
            pltpu.VMEM((N, 16), jnp.float32),
            pltpu.VMEM((N, 16), jnp.float32),
        ],
    )
    return kern(p1_feat, p2_feat, n1_feat, n2_feat)


def _tc_body(p1_ref, p2_ref, n1_ref, n2_ref, out_ref, d2_ref, d1_ref, *, inv_count):
    i = pl.program_id(0)

    @pl.when(i == 0)
    def _init():
        out_ref[...] = jnp.zeros((1, 1), jnp.float32)

    p1 = p1_ref[...][:, None, :]
    p2 = p2_ref[...][:, None, :]
    d2_ref[...] = jnp.sum(p1 * (n2_ref[...] - p2), axis=-1)
    d1_ref[...] = jnp.sum(p2 * (n1_ref[...] - p1), axis=-1)
    d2 = d2_ref[...]
    d1 = d1_ref[...]

    def softplus(x):
        return jnp.maximum(x, 0.0) + jnp.log1p(jnp.exp(-jnp.abs(x)))

    partial = jnp.sum(softplus(d2)) + jnp.sum(softplus(d1))
    out_ref[...] += (partial * inv_count).reshape(1, 1)


def _tc_main(p1_feat, p2_feat, n1_feat, n2_feat, inv_count):
    B, N, D = n1_feat.shape
    off = SC_ROWS // TC_BLK
    grid = (B - SC_ROWS) // TC_BLK
    return pl.pallas_call(
        functools.partial(_tc_body, inv_count=inv_count),
        grid=(grid,),
        in_specs=[
            pl.BlockSpec((TC_BLK, D), lambda i: (i + off, 0)),
            pl.BlockSpec((TC_BLK, D), lambda i: (i + off, 0)),
            pl.BlockSpec((TC_BLK, N, D), lambda i: (i + off, 0, 0)),
            pl.BlockSpec((TC_BLK, N, D), lambda i: (i + off, 0, 0)),
        ],
        out_specs=pl.BlockSpec((1, 1), lambda i: (0, 0)),
        out_shape=jax.ShapeDtypeStruct((1, 1), jnp.float32),
        scratch_shapes=[
            pltpu.VMEM((TC_BLK, N), jnp.float32),
            pltpu.VMEM((TC_BLK, N), jnp.float32),
        ],
    )(p1_feat, p2_feat, n1_feat, n2_feat)


def _epi_body(d2_ref, d1_ref, part_ref, out_ref, *, inv_count, n_neg):
    k = d2_ref.shape[1]
    seg = jax.lax.broadcasted_iota(jnp.int32, (k, n_neg), 0) // 16
    col = jax.lax.broadcasted_iota(jnp.int32, (k, n_neg), 1)
    m = (seg == col).astype(jnp.float32)
    s2 = jnp.dot(d2_ref[...], m, preferred_element_type=jnp.float32)
    s1 = jnp.dot(d1_ref[...], m, preferred_element_type=jnp.float32)

    def softplus(x):
        return jnp.maximum(x, 0.0) + jnp.log1p(jnp.exp(-jnp.abs(x)))

    partial = jnp.sum(softplus(s2)) + jnp.sum(softplus(s1))
    out_ref[...] = part_ref[...] + (partial * inv_count).reshape(1, 1)


def _epilogue(d2a, d1a, part_main, inv_count):
    S, N, L = d2a.shape
    d2f = d2a.reshape(S, N * L)
    d1f = d1a.reshape(S, N * L)
    return pl.pallas_call(
        functools.partial(_epi_body, inv_count=inv_count, n_neg=N),
        in_specs=[
            pl.BlockSpec((S, N * L), lambda: (0, 0)),
            pl.BlockSpec((S, N * L), lambda: (0, 0)),
            pl.BlockSpec((1, 1), lambda: (0, 0)),
        ],
        out_specs=pl.BlockSpec((1, 1), lambda: (0, 0)),
        out_shape=jax.ShapeDtypeStruct((1, 1), jnp.float32),
    )(d2f, d1f, part_main)


def kernel(p1_feat, p2_feat, n1_feat, n2_feat, relation):
    B, N, D = n1_feat.shape
    inv_count = 1.0 / (B * N)
    d2a, d1a = _sc_scores(p1_feat, p2_feat, n1_feat, n2_feat)
    part_main = _tc_main(p1_feat, p2_feat, n1_feat, n2_feat, inv_count)
    out = _epilogue(d2a, d1a, part_main, inv_count)
    return out[0, 0]


# speedup vs baseline: 1.0982x; 1.0982x over previous
import functools

import jax
import jax.numpy as jnp
from jax import lax
from jax.experimental import pallas as pl
from jax.experimental.pallas import tpu as pltpu
from jax.experimental.pallas import tpu_sc as plsc

SC_ROWS = 256
TC_BLK = 256


def _sc_body(p1_hbm, p2_hbm, n1_hbm, n2_hbm, d2_hbm, d1_hbm,
             p1_v, p2_v, n1_v, n2_v, d2_v, d1_v, *, rpw, n_neg, d_dim):
    nc = 2
    wid = lax.axis_index("s") * nc + lax.axis_index("c")
    base = wid * rpw

    pltpu.sync_copy(p1_hbm.at[pl.ds(base, rpw)], p1_v)
    pltpu.sync_copy(p2_hbm.at[pl.ds(base, rpw)], p2_v)

    nchunks = d_dim // 16

    def row_step(r, carry):
        row = base + r
        pltpu.sync_copy(n2_hbm.at[row], n2_v)
        pltpu.sync_copy(n1_hbm.at[row], n1_v)
        p1c = [p1_v.at[r][pl.ds(c * 16, 16)][...] for c in range(nchunks)]
        p2c = [p2_v.at[r][pl.ds(c * 16, 16)][...] for c in range(nchunks)]
        for n in range(n_neg):
            acc2 = p1c[0] * (n2_v.at[n][pl.ds(0, 16)][...] - p2c[0])
            acc1 = p2c[0] * (n1_v.at[n][pl.ds(0, 16)][...] - p1c[0])
            for c in range(1, nchunks):
                acc2 += p1c[c] * (n2_v.at[n][pl.ds(c * 16, 16)][...] - p2c[c])
                acc1 += p2c[c] * (n1_v.at[n][pl.ds(c * 16, 16)][...] - p1c[c])
            d2_v[n] = acc2
            d1_v[n] = acc1
        pltpu.sync_copy(d2_v, d2_hbm.at[row])
        pltpu.sync_copy(d1_v, d1_hbm.at[row])
        return carry

    lax.fori_loop(0, rpw, row_step, 0)


def _sc_scores(p1_feat, p2_feat, n1_feat, n2_feat):
    B, N, D = n1_feat.shape
    info = plsc.get_sparse_core_info()
    nw = info.num_cores * info.num_subcores
    rpw = SC_ROWS // nw
    mesh = plsc.VectorSubcoreMesh(core_axis_name="c", subcore_axis_name="s")
    kern = pl.kernel(
        functools.partial(_sc_body, rpw=rpw, n_neg=N, d_dim=D),
        out_type=(
            jax.ShapeDtypeStruct((SC_ROWS, N, 16), jnp.float32),
            jax.ShapeDtypeStruct((SC_ROWS, N, 16), jnp.float32),
        ),
        mesh=mesh,
        scratch_types=[
            pltpu.VMEM((rpw, D), jnp.float32),
            pltpu.VMEM((rpw, D), jnp.float32),
            pltpu.VMEM((N, D), jnp.float32),
            pltpu.VMEM((N, D), jnp.float32),
            pltpu.VMEM((N, 16), jnp.float32),
            pltpu.VMEM((N, 16), jnp.float32),
        ],
    )
    return kern(p1_feat, p2_feat, n1_feat, n2_feat)


def _tc_body(p1_ref, p2_ref, n1_ref, n2_ref, out_ref, d2_ref, d1_ref, *, inv_count):
    i = pl.program_id(0)

    @pl.when(i == 0)
    def _init():
        out_ref[...] = jnp.zeros((1, 1), jnp.float32)

    p1 = p1_ref[...][:, None, :]
    p2 = p2_ref[...][:, None, :]
    d2_ref[...] = jnp.sum(p1 * (n2_ref[...] - p2), axis=-1)
    d1_ref[...] = jnp.sum(p2 * (n1_ref[...] - p1), axis=-1)
    d2 = d2_ref[...]
    d1 = d1_ref[...]

    def softplus(x):
        return jnp.maximum(x, 0.0) + jnp.log1p(jnp.exp(-jnp.abs(x)))

    partial = jnp.sum(softplus(d2)) + jnp.sum(softplus(d1))
    out_ref[...] += (partial * inv_count).reshape(1, 1)


def _tc_main(p1_feat, p2_feat, n1_feat, n2_feat, inv_count):
    B, N, D = n1_feat.shape
    off = SC_ROWS // TC_BLK
    grid = (B - SC_ROWS) // TC_BLK
    return pl.pallas_call(
        functools.partial(_tc_body, inv_count=inv_count),
        grid=(grid,),
        in_specs=[
            pl.BlockSpec((TC_BLK, D), lambda i: (i + off, 0)),
            pl.BlockSpec((TC_BLK, D), lambda i: (i + off, 0)),
            pl.BlockSpec((TC_BLK, N, D), lambda i: (i + off, 0, 0)),
            pl.BlockSpec((TC_BLK, N, D), lambda i: (i + off, 0, 0)),
        ],
        out_specs=pl.BlockSpec((1, 1), lambda i: (0, 0)),
        out_shape=jax.ShapeDtypeStruct((1, 1), jnp.float32),
        scratch_shapes=[
            pltpu.VMEM((TC_BLK, N), jnp.float32),
            pltpu.VMEM((TC_BLK, N), jnp.float32),
        ],
    )(p1_feat, p2_feat, n1_feat, n2_feat)


def _epi_body(d2_ref, d1_ref, part_ref, out_ref, *, inv_count, n_neg):
    k = d2_ref.shape[1]
    seg = jax.lax.broadcasted_iota(jnp.int32, (k, n_neg), 0) // 16
    col = jax.lax.broadcasted_iota(jnp.int32, (k, n_neg), 1)
    m = (seg == col).astype(jnp.float32)
    s2 = jnp.dot(d2_ref[...], m, preferred_element_type=jnp.float32)
    s1 = jnp.dot(d1_ref[...], m, preferred_element_type=jnp.float32)

    def softplus(x):
        return jnp.maximum(x, 0.0) + jnp.log1p(jnp.exp(-jnp.abs(x)))

    partial = jnp.sum(softplus(s2)) + jnp.sum(softplus(s1))
    out_ref[...] = part_ref[...] + (partial * inv_count).reshape(1, 1)


def _epilogue(d2a, d1a, part_main, inv_count):
    S, N, L = d2a.shape
    d2f = d2a.reshape(S, N * L)
    d1f = d1a.reshape(S, N * L)
    return pl.pallas_call(
        functools.partial(_epi_body, inv_count=inv_count, n_neg=N),
        in_specs=[
            pl.BlockSpec((S, N * L), lambda: (0, 0)),
            pl.BlockSpec((S, N * L), lambda: (0, 0)),
            pl.BlockSpec((1, 1), lambda: (0, 0)),
        ],
        out_specs=pl.BlockSpec((1, 1), lambda: (0, 0)),
        out_shape=jax.ShapeDtypeStruct((1, 1), jnp.float32),
    )(d2f, d1f, part_main)


def kernel(p1_feat, p2_feat, n1_feat, n2_feat, relation):
    B, N, D = n1_feat.shape
    inv_count = 1.0 / (B * N)
    d2a, d1a = _sc_scores(p1_feat, p2_feat, n1_feat, n2_feat)
    part_main = _tc_main(p1_feat, p2_feat, n1_feat, n2_feat, inv_count)
    out = _epilogue(d2a, d1a, part_main, inv_count)
    return out[0, 0]


# speedup vs baseline: 1.5506x; 1.4119x over previous
import functools

import jax
import jax.numpy as jnp
from jax.experimental import pallas as pl
from jax.experimental.pallas import tpu as pltpu


def _body(p1_ref, p2_ref, n1_ref, n2_ref, out_ref, d_ref, *, inv_count, steps):
    i = pl.program_id(0)

    @pl.when(i == 0)
    def _init():
        out_ref[...] = jnp.zeros((1, 1), jnp.float32)

    @pl.when(i > 0)
    def _softplus_prev():
        def softplus(x):
            return jnp.maximum(x, 0.0) + jnp.log1p(jnp.exp(-jnp.abs(x)))

        partial = jnp.sum(softplus(d_ref[...]))
        out_ref[...] += (partial * inv_count).reshape(1, 1)

    @pl.when(i < steps)
    def _reduce_cur():
        p1 = p1_ref[...][:, None, :]
        p2 = p2_ref[...][:, None, :]
        n = n2_ref.shape[1]
        d_ref[:, :n] = jnp.sum(p1 * (n2_ref[...] - p2), axis=-1)
        d_ref[:, n:] = jnp.sum(p2 * (n1_ref[...] - p1), axis=-1)


def kernel(p1_feat, p2_feat, n1_feat, n2_feat, relation):
    B, N, D = n1_feat.shape
    BLK = 256
    grid = B // BLK
    inv_count = 1.0 / (B * N)

    out = pl.pallas_call(
        functools.partial(_body, inv_count=inv_count, steps=grid),
        grid=(grid + 1,),
        in_specs=[
            pl.BlockSpec((BLK, D), lambda i: (jnp.minimum(i, 15), 0)),
            pl.BlockSpec((BLK, D), lambda i: (jnp.minimum(i, 15), 0)),
            pl.BlockSpec((BLK, N, D), lambda i: (jnp.minimum(i, 15), 0, 0)),
            pl.BlockSpec((BLK, N, D), lambda i: (jnp.minimum(i, 15), 0, 0)),
        ],
        out_specs=pl.BlockSpec((1, 1), lambda i: (0, 0)),
        out_shape=jax.ShapeDtypeStruct((1, 1), jnp.float32),
        scratch_shapes=[
            pltpu.VMEM((BLK, 2 * N), jnp.float32),
        ],
        compiler_params=pltpu.CompilerParams(
            vmem_limit_bytes=128 * 1024 * 1024,
        ),
    )(p1_feat, p2_feat, n1_feat, n2_feat)
    return out[0, 0]
